# Initial kernel scaffold; baseline (speedup 1.0000x reference)
import functools

import jax
import jax.numpy as jnp
from jax import lax
from jax.experimental import pallas as pl
from jax.experimental.pallas import tpu as pltpu
from jax.experimental.pallas import tpu_sc as plsc

N = 10000
E = 320000
D = 128

NC = 2
NS = 16
LANES = 16
NW = NC * NS

CHUNK = 128
E_PAD = ((E + NW * CHUNK - 1) // (NW * CHUNK)) * (NW * CHUNK)
EPT = E_PAD // NW
NCH = EPT // CHUNK

N_PAD = 10240
ROWS_PT = N_PAD // NS
ZCOPIES = ROWS_PT // CHUNK

_mesh = plsc.VectorSubcoreMesh(core_axis_name="c", subcore_axis_name="s")


def _sc_degree(dst3, ones_l, zeros_l):
    @functools.partial(
        pl.kernel,
        out_type=jax.ShapeDtypeStruct((NC, N_PAD, LANES), jnp.float32),
        mesh=_mesh,
        scratch_types=[
            pltpu.VMEM((NCH, CHUNK), jnp.int32),
            pltpu.VMEM((CHUNK, LANES), jnp.float32),
            pltpu.VMEM((ROWS_PT, LANES), jnp.float32),
            pltpu.VMEM_SHARED((N_PAD, LANES), jnp.float32),
        ],
    )
    def deg_kernel(dst_hbm, ones_hbm, zer_hbm, deg_hbm, dst_v, ones_v, zer_v, acc):
        c = lax.axis_index("c")
        s = lax.axis_index("s")
        w = c * NS + s
        pltpu.sync_copy(dst_hbm.at[w], dst_v)
        pltpu.sync_copy(ones_hbm, ones_v)
        pltpu.sync_copy(zer_hbm, zer_v)
        pltpu.sync_copy(zer_v, acc.at[pl.ds(s * ROWS_PT, ROWS_PT)])
        plsc.subcore_barrier()

        @pl.loop(0, NCH)
        def _(j):
            pltpu.sync_copy(ones_v, acc.at[dst_v.at[j]], add=True)

        plsc.subcore_barrier()
        pltpu.sync_copy(
            acc.at[pl.ds(s * ROWS_PT, ROWS_PT)],
            deg_hbm.at[c, pl.ds(s * ROWS_PT, ROWS_PT)],
        )

    return deg_kernel(dst3, ones_l, zeros_l)


def _sc_scatter(g, src3, dst3, zeros_d):
    @functools.partial(
        pl.kernel,
        out_type=jax.ShapeDtypeStruct((NC, N_PAD, D), jnp.float32),
        mesh=_mesh,
        scratch_types=[
            pltpu.VMEM((NCH, CHUNK), jnp.int32),
            pltpu.VMEM((NCH, CHUNK), jnp.int32),
            pltpu.VMEM((CHUNK, D), jnp.float32),
            pltpu.VMEM_SHARED((N_PAD, D), jnp.float32),
        ],
    )
    def scat_kernel(g_hbm, src_hbm, dst_hbm, zer_hbm, out_hbm, src_v, dst_v, buf, acc):
        c = lax.axis_index("c")
        s = lax.axis_index("s")
        w = c * NS + s
        pltpu.sync_copy(src_hbm.at[w], src_v)
        pltpu.sync_copy(dst_hbm.at[w], dst_v)
        pltpu.sync_copy(zer_hbm, buf)

        @pl.loop(0, ZCOPIES)
        def _(k):
            pltpu.sync_copy(buf, acc.at[pl.ds(s * ROWS_PT + k * CHUNK, CHUNK)])

        plsc.subcore_barrier()

        @pl.loop(0, NCH)
        def _(j):
            pltpu.sync_copy(g_hbm.at[src_v.at[j]], buf)
            pltpu.sync_copy(buf, acc.at[dst_v.at[j]], add=True)

        plsc.subcore_barrier()

        @pl.loop(0, ZCOPIES)
        def _(k):
            r = s * ROWS_PT + k * CHUNK
            pltpu.sync_copy(acc.at[pl.ds(r, CHUNK)], out_hbm.at[c, pl.ds(r, CHUNK)])

    return scat_kernel(g, src3, dst3, zeros_d)


_R = 1024
_GRID = N_PAD // _R


def _dinv_block(deg_ref):
    deg = deg_ref[0, :, 0:1] + deg_ref[1, :, 0:1] + 1.0
    return lax.rsqrt(deg)


def _row_mask(i):
    rid = i * _R + lax.broadcasted_iota(jnp.int32, (_R, 1), 0)
    return rid < N


def _first_body(x_ref, deg_ref, w_ref, o_ref):
    i = pl.program_id(0)
    dinv = _dinv_block(deg_ref)
    h = jnp.dot(x_ref[...], w_ref[...], preferred_element_type=jnp.float32)
    o_ref[...] = jnp.where(_row_mask(i), dinv * h, 0.0)


def _tc_first(x_p, degp, W1):
    return pl.pallas_call(
        _first_body,
        grid=(_GRID,),
        in_specs=[
            pl.BlockSpec((_R, D), lambda i: (i, 0)),
            pl.BlockSpec((NC, _R, LANES), lambda i: (0, i, 0)),
            pl.BlockSpec((D, D), lambda i: (0, 0)),
        ],
        out_specs=pl.BlockSpec((_R, D), lambda i: (i, 0)),
        out_shape=jax.ShapeDtypeStruct((N_PAD, D), jnp.float32),
    )(x_p, degp, W1)


def _mid_body(p_ref, g_ref, deg_ref, b_ref, w_ref, o_ref):
    i = pl.program_id(0)
    dinv = _dinv_block(deg_ref)
    z = dinv * (p_ref[0] + p_ref[1] + g_ref[...]) + b_ref[...]
    x2 = jnp.maximum(z, 0.0)
    h = jnp.dot(x2, w_ref[...], preferred_element_type=jnp.float32)
    o_ref[...] = jnp.where(_row_mask(i), dinv * h, 0.0)


def _tc_mid(P, g, degp, b, W):
    return pl.pallas_call(
        _mid_body,
        grid=(_GRID,),
        in_specs=[
            pl.BlockSpec((NC, _R, D), lambda i: (0, i, 0)),
            pl.BlockSpec((_R, D), lambda i: (i, 0)),
            pl.BlockSpec((NC, _R, LANES), lambda i: (0, i, 0)),
            pl.BlockSpec((1, D), lambda i: (0, 0)),
            pl.BlockSpec((D, D), lambda i: (0, 0)),
        ],
        out_specs=pl.BlockSpec((_R, D), lambda i: (i, 0)),
        out_shape=jax.ShapeDtypeStruct((N_PAD, D), jnp.float32),
    )(P, g, degp, b.reshape(1, D), W)


def _last_body(p_ref, g_ref, deg_ref, b_ref, w_ref, bo_ref, o_ref):
    dinv = _dinv_block(deg_ref)
    z = dinv * (p_ref[0] + p_ref[1] + g_ref[...]) + b_ref[...]
    x3 = jnp.maximum(z, 0.0)
    h = jnp.dot(x3, w_ref[...], preferred_element_type=jnp.float32)
    o_ref[...] = h + bo_ref[...]


def _tc_last(Q, g, degp, b, Wfc, bfc):
    return pl.pallas_call(
        _last_body,
        grid=(_GRID,),
        in_specs=[
            pl.BlockSpec((NC, _R, D), lambda i: (0, i, 0)),
            pl.BlockSpec((_R, D), lambda i: (i, 0)),
            pl.BlockSpec((NC, _R, LANES), lambda i: (0, i, 0)),
            pl.BlockSpec((1, D), lambda i: (0, 0)),
            pl.BlockSpec((D, D), lambda i: (0, 0)),
            pl.BlockSpec((1, D), lambda i: (0, 0)),
        ],
        out_specs=pl.BlockSpec((_R, D), lambda i: (i, 0)),
        out_shape=jax.ShapeDtypeStruct((N_PAD, D), jnp.float32),
    )(Q, g, degp, b.reshape(1, D), Wfc, bfc.reshape(1, D))


def kernel(x, edge_index, W1, b1, W2, b2, Wfc, bfc):
    src = edge_index[0]
    dst = edge_index[1]
    pad = E_PAD - E
    pad_idx = jnp.full((pad,), N, dtype=jnp.int32)
    src3 = jnp.concatenate([src, pad_idx]).reshape(NW, NCH, CHUNK)
    dst3 = jnp.concatenate([dst, pad_idx]).reshape(NW, NCH, CHUNK)

    x_p = jnp.pad(x, ((0, N_PAD - N), (0, 0)))
    ones_l = jnp.ones((CHUNK, LANES), jnp.float32)
    zeros_l = jnp.zeros((ROWS_PT, LANES), jnp.float32)
    zeros_d = jnp.zeros((CHUNK, D), jnp.float32)

    degp = _sc_degree(dst3, ones_l, zeros_l)
    g1 = _tc_first(x_p, degp, W1)
    P = _sc_scatter(g1, src3, dst3, zeros_d)
    g2 = _tc_mid(P, g1, degp, b1, W2)
    Q = _sc_scatter(g2, src3, dst3, zeros_d)
    out = _tc_last(Q, g2, degp, b2, Wfc, bfc)
    return out[:N]


# speedup vs baseline: 10.6475x; 10.6475x over previous
import functools

import jax
import jax.numpy as jnp
from jax import lax
from jax.experimental import pallas as pl
from jax.experimental.pallas import tpu as pltpu
from jax.experimental.pallas import tpu_sc as plsc

N = 10000
E = 320000
D = 128

NC = 2
NS = 16
NW = NC * NS

CHUNK = 128
E_PAD = ((E + NW * CHUNK - 1) // (NW * CHUNK)) * (NW * CHUNK)
EPT = E_PAD // NW
NCH = EPT // CHUNK

N_PAD = 10240

_mesh = plsc.VectorSubcoreMesh(core_axis_name="c", subcore_axis_name="s")


def _sc_degree(dst3, ones_l, zeros_big):
    @functools.partial(
        pl.kernel,
        out_type=jax.ShapeDtypeStruct((NC, N_PAD, D), jnp.float32),
        mesh=_mesh,
        scratch_types=[
            pltpu.VMEM((CHUNK,), jnp.int32),
            pltpu.VMEM((CHUNK, D), jnp.float32),
            pltpu.VMEM_SHARED((N_PAD, D), jnp.float32),
        ],
    )
    def deg_kernel(dst_hbm, ones_hbm, zer_hbm, deg_hbm, dst_i, ones_v, acc):
        c = lax.axis_index("c")
        s = lax.axis_index("s")
        w = c * NS + s
        pltpu.sync_copy(ones_hbm, ones_v)

        @pl.when(s == 0)
        def _():
            pltpu.sync_copy(zer_hbm, acc)

        plsc.subcore_barrier()

        @pl.loop(0, NCH)
        def _(j):
            pltpu.sync_copy(dst_hbm.at[w, j], dst_i)
            pltpu.sync_copy(ones_v, acc.at[dst_i], add=True)

        plsc.subcore_barrier()

        @pl.when(s == 0)
        def _():
            pltpu.sync_copy(acc, deg_hbm.at[c])

    return deg_kernel(dst3, ones_l, zeros_big)


def _sc_scatter(g, src3, dst3, zeros_big):
    @functools.partial(
        pl.kernel,
        out_type=jax.ShapeDtypeStruct((NC, N_PAD, D), jnp.float32),
        mesh=_mesh,
        scratch_types=[
            pltpu.VMEM((CHUNK,), jnp.int32),
            pltpu.VMEM((CHUNK,), jnp.int32),
            pltpu.VMEM((CHUNK, D), jnp.float32),
            pltpu.VMEM_SHARED((N_PAD, D), jnp.float32),
        ],
    )
    def scat_kernel(g_hbm, src_hbm, dst_hbm, zer_hbm, out_hbm, src_i, dst_i, buf, acc):
        c = lax.axis_index("c")
        s = lax.axis_index("s")
        w = c * NS + s

        @pl.when(s == 0)
        def _():
            pltpu.sync_copy(zer_hbm, acc)

        plsc.subcore_barrier()

        @pl.loop(0, NCH)
        def _(j):
            pltpu.sync_copy(src_hbm.at[w, j], src_i)
            pltpu.sync_copy(g_hbm.at[src_i], buf)
            pltpu.sync_copy(dst_hbm.at[w, j], dst_i)
            pltpu.sync_copy(buf, acc.at[dst_i], add=True)

        plsc.subcore_barrier()

        @pl.when(s == 0)
        def _():
            pltpu.sync_copy(acc, out_hbm.at[c])

    return scat_kernel(g, src3, dst3, zeros_big)


_R = 1024
_GRID = N_PAD // _R


def _dinv_block(deg_ref):
    deg = deg_ref[0, :, 0:1] + deg_ref[1, :, 0:1] + 1.0
    return lax.rsqrt(deg)


def _row_mask(i):
    rid = i * _R + lax.broadcasted_iota(jnp.int32, (_R, 1), 0)
    return rid < N


def _first_body(x_ref, deg_ref, w_ref, o_ref):
    i = pl.program_id(0)
    dinv = _dinv_block(deg_ref)
    h = jnp.dot(x_ref[...], w_ref[...], preferred_element_type=jnp.float32)
    o_ref[...] = jnp.where(_row_mask(i), dinv * h, 0.0)


def _tc_first(x_p, degp, W1):
    return pl.pallas_call(
        _first_body,
        grid=(_GRID,),
        in_specs=[
            pl.BlockSpec((_R, D), lambda i: (i, 0)),
            pl.BlockSpec((NC, _R, D), lambda i: (0, i, 0)),
            pl.BlockSpec((D, D), lambda i: (0, 0)),
        ],
        out_specs=pl.BlockSpec((_R, D), lambda i: (i, 0)),
        out_shape=jax.ShapeDtypeStruct((N_PAD, D), jnp.float32),
    )(x_p, degp, W1)


def _mid_body(p_ref, g_ref, deg_ref, b_ref, w_ref, o_ref):
    i = pl.program_id(0)
    dinv = _dinv_block(deg_ref)
    z = dinv * (p_ref[0] + p_ref[1] + g_ref[...]) + b_ref[...]
    x2 = jnp.maximum(z, 0.0)
    h = jnp.dot(x2, w_ref[...], preferred_element_type=jnp.float32)
    o_ref[...] = jnp.where(_row_mask(i), dinv * h, 0.0)


def _tc_mid(P, g, degp, b, W):
    return pl.pallas_call(
        _mid_body,
        grid=(_GRID,),
        in_specs=[
            pl.BlockSpec((NC, _R, D), lambda i: (0, i, 0)),
            pl.BlockSpec((_R, D), lambda i: (i, 0)),
            pl.BlockSpec((NC, _R, D), lambda i: (0, i, 0)),
            pl.BlockSpec((1, D), lambda i: (0, 0)),
            pl.BlockSpec((D, D), lambda i: (0, 0)),
        ],
        out_specs=pl.BlockSpec((_R, D), lambda i: (i, 0)),
        out_shape=jax.ShapeDtypeStruct((N_PAD, D), jnp.float32),
    )(P, g, degp, b.reshape(1, D), W)


def _last_body(p_ref, g_ref, deg_ref, b_ref, w_ref, bo_ref, o_ref):
    dinv = _dinv_block(deg_ref)
    z = dinv * (p_ref[0] + p_ref[1] + g_ref[...]) + b_ref[...]
    x3 = jnp.maximum(z, 0.0)
    h = jnp.dot(x3, w_ref[...], preferred_element_type=jnp.float32)
    o_ref[...] = h + bo_ref[...]


def _tc_last(Q, g, degp, b, Wfc, bfc):
    return pl.pallas_call(
        _last_body,
        grid=(_GRID,),
        in_specs=[
            pl.BlockSpec((NC, _R, D), lambda i: (0, i, 0)),
            pl.BlockSpec((_R, D), lambda i: (i, 0)),
            pl.BlockSpec((NC, _R, D), lambda i: (0, i, 0)),
            pl.BlockSpec((1, D), lambda i: (0, 0)),
            pl.BlockSpec((D, D), lambda i: (0, 0)),
            pl.BlockSpec((1, D), lambda i: (0, 0)),
        ],
        out_specs=pl.BlockSpec((_R, D), lambda i: (i, 0)),
        out_shape=jax.ShapeDtypeStruct((N_PAD, D), jnp.float32),
    )(Q, g, degp, b.reshape(1, D), Wfc, bfc.reshape(1, D))


def kernel(x, edge_index, W1, b1, W2, b2, Wfc, bfc):
    src = edge_index[0]
    dst = edge_index[1]
    pad = E_PAD - E
    pad_idx = jnp.full((pad,), N, dtype=jnp.int32)
    src3 = jnp.concatenate([src, pad_idx]).reshape(NW, NCH, CHUNK)
    dst3 = jnp.concatenate([dst, pad_idx]).reshape(NW, NCH, CHUNK)

    x_p = jnp.pad(x, ((0, N_PAD - N), (0, 0)))
    ones_l = jnp.ones((CHUNK, D), jnp.float32)
    zeros_big = jnp.zeros((N_PAD, D), jnp.float32)

    degp = _sc_degree(dst3, ones_l, zeros_big)
    g1 = _tc_first(x_p, degp, W1)
    P = _sc_scatter(g1, src3, dst3, zeros_big)
    g2 = _tc_mid(P, g1, degp, b1, W2)
    Q = _sc_scatter(g2, src3, dst3, zeros_big)
    out = _tc_last(Q, g2, degp, b2, Wfc, bfc)
    return out[:N]
